# Initial kernel scaffold; baseline (speedup 1.0000x reference)
import jax
import jax.numpy as jnp
from jax.experimental import pallas as pl


def kernel(pred_sg_up, edge_v, adj):
    raise NotImplementedError("write your pallas kernel here")


# speedup vs baseline: 2.6930x; 2.6930x over previous
import functools

import jax
import jax.numpy as jnp
from jax.experimental import pallas as pl

_C = 21
_L = 256
_K = 252


def _seg_stats_kernel(seg_ref, edge_ref, stats_ref):
    j = pl.program_id(1)
    x = seg_ref[0]
    m = jnp.max(x, axis=0, keepdims=True)
    e = jnp.exp(x - m)
    s = jnp.sum(e, axis=0, keepdims=True)
    p = e / s
    pe = jnp.concatenate([p, jnp.ones((1, p.shape[1]), p.dtype)], axis=0)

    labels = edge_ref[0, 0]
    iota = jax.lax.broadcasted_iota(jnp.int32, (labels.shape[0], _L), 1)
    onehot = (iota == labels).astype(jnp.float32)

    part = jax.lax.dot_general(
        pe, onehot, (((1,), (0,)), ((), ())),
        preferred_element_type=jnp.float32,
        precision=jax.lax.Precision.HIGHEST)

    @pl.when(j == 0)
    def _init():
        stats_ref[0] = part

    @pl.when(j > 0)
    def _acc():
        stats_ref[0] += part


def _tail_kernel(stats_ref, adj_ref, out_ref, *, nb):
    b = pl.program_id(0)
    stats = stats_ref[0]
    sums = stats[:_C, :_K]
    counts = stats[_C:, :_K]
    safe = jnp.where(counts > 0, counts, jnp.ones_like(counts))
    mu = jnp.where(counts > 0, sums / safe, jnp.zeros_like(sums))
    nrm = jnp.sqrt(jnp.sum(mu * mu, axis=0, keepdims=True))
    mun = mu / (nrm + 1e-6)
    gram = jax.lax.dot_general(
        mun, mun, (((0,), (0,)), ((), ())),
        preferred_element_type=jnp.float32,
        precision=jax.lax.Precision.HIGHEST)
    adj = adj_ref[0]
    num = jnp.sum(gram * adj, axis=1, keepdims=True)
    den = jnp.sum(adj, axis=1, keepdims=True) + 1e-6
    part = jnp.sum(num / den, axis=(0, 1), keepdims=True) / (nb * _K)

    @pl.when(b == 0)
    def _init():
        out_ref[...] = part

    @pl.when(b > 0)
    def _acc():
        out_ref[...] += part


def kernel(pred_sg_up, edge_v, adj):
    B, C, H, W = pred_sg_up.shape
    N = H * W
    bN = 2048
    nblk = N // bN
    seg = pred_sg_up.reshape(B, C, N)
    edge = edge_v.reshape(B, nblk, bN, 1)

    stats = pl.pallas_call(
        _seg_stats_kernel,
        grid=(B, nblk),
        in_specs=[
            pl.BlockSpec((1, C, bN), lambda b, j: (b, 0, j)),
            pl.BlockSpec((1, 1, bN, 1), lambda b, j: (b, j, 0, 0)),
        ],
        out_specs=pl.BlockSpec((1, C + 1, _L), lambda b, j: (b, 0, 0)),
        out_shape=jax.ShapeDtypeStruct((B, C + 1, _L), jnp.float32),
    )(seg, edge)

    loss = pl.pallas_call(
        functools.partial(_tail_kernel, nb=B),
        grid=(B,),
        in_specs=[
            pl.BlockSpec((1, C + 1, _L), lambda b: (b, 0, 0)),
            pl.BlockSpec((1, _K, _K), lambda b: (b, 0, 0)),
        ],
        out_specs=pl.BlockSpec((1, 1), lambda b: (0, 0)),
        out_shape=jax.ShapeDtypeStruct((1, 1), jnp.float32),
    )(stats, adj)
    return loss[0, 0]


# speedup vs baseline: 3.2727x; 1.2153x over previous
import functools

import jax
import jax.numpy as jnp
from jax.experimental import pallas as pl

_C = 21
_L = 256
_K = 252


def _seg_stats_kernel(seg_ref, edge_ref, stats_ref):
    j = pl.program_id(1)
    x = seg_ref[0]
    m = jnp.max(x, axis=0, keepdims=True)
    e = jnp.exp(x - m)
    s = jnp.sum(e, axis=0, keepdims=True)
    p = e / s
    pe = jnp.concatenate([p, jnp.ones((1, p.shape[1]), p.dtype)], axis=0)
    pe = pe.astype(jnp.bfloat16)

    labels = edge_ref[0, 0]
    iota = jax.lax.broadcasted_iota(jnp.int32, (labels.shape[0], _L), 1)
    onehot = (iota == labels).astype(jnp.bfloat16)

    part = jax.lax.dot_general(
        pe, onehot, (((1,), (0,)), ((), ())),
        preferred_element_type=jnp.float32)

    @pl.when(j == 0)
    def _init():
        stats_ref[0] = part

    @pl.when(j > 0)
    def _acc():
        stats_ref[0] += part


def _tail_kernel(stats_ref, adj_ref, out_ref, *, nb):
    b = pl.program_id(0)
    stats = stats_ref[0]
    sums = stats[:_C, :_K]
    counts = stats[_C:, :_K]
    safe = jnp.where(counts > 0, counts, jnp.ones_like(counts))
    mu = jnp.where(counts > 0, sums / safe, jnp.zeros_like(sums))
    nrm = jnp.sqrt(jnp.sum(mu * mu, axis=0, keepdims=True))
    mun = mu / (nrm + 1e-6)
    gram = jax.lax.dot_general(
        mun, mun, (((0,), (0,)), ((), ())),
        preferred_element_type=jnp.float32,
        precision=jax.lax.Precision.HIGHEST)
    adj = adj_ref[0]
    num = jnp.sum(gram * adj, axis=1, keepdims=True)
    den = jnp.sum(adj, axis=1, keepdims=True) + 1e-6
    part = jnp.sum(num / den, axis=(0, 1), keepdims=True) / (nb * _K)

    @pl.when(b == 0)
    def _init():
        out_ref[...] = part

    @pl.when(b > 0)
    def _acc():
        out_ref[...] += part


def kernel(pred_sg_up, edge_v, adj):
    B, C, H, W = pred_sg_up.shape
    N = H * W
    bN = 2048
    nblk = N // bN
    seg = pred_sg_up.reshape(B, C, N)
    edge = edge_v.reshape(B, nblk, bN, 1)

    stats = pl.pallas_call(
        _seg_stats_kernel,
        grid=(B, nblk),
        in_specs=[
            pl.BlockSpec((1, C, bN), lambda b, j: (b, 0, j)),
            pl.BlockSpec((1, 1, bN, 1), lambda b, j: (b, j, 0, 0)),
        ],
        out_specs=pl.BlockSpec((1, C + 1, _L), lambda b, j: (b, 0, 0)),
        out_shape=jax.ShapeDtypeStruct((B, C + 1, _L), jnp.float32),
    )(seg, edge)

    loss = pl.pallas_call(
        functools.partial(_tail_kernel, nb=B),
        grid=(B,),
        in_specs=[
            pl.BlockSpec((1, C + 1, _L), lambda b: (b, 0, 0)),
            pl.BlockSpec((1, _K, _K), lambda b: (b, 0, 0)),
        ],
        out_specs=pl.BlockSpec((1, 1), lambda b: (0, 0)),
        out_shape=jax.ShapeDtypeStruct((1, 1), jnp.float32),
    )(stats, adj)
    return loss[0, 0]


# speedup vs baseline: 4.1530x; 1.2690x over previous
import functools

import jax
import jax.numpy as jnp
from jax.experimental import pallas as pl

_C = 21
_L = 256
_K = 252


def _seg_stats_kernel(seg_ref, edge_ref, stats_ref):
    j = pl.program_id(1)
    x = seg_ref[0]
    m = jnp.max(x, axis=0, keepdims=True)
    e = jnp.exp(x - m)
    s = jnp.sum(e, axis=0, keepdims=True)
    p = e / s
    pe = jnp.concatenate([p, jnp.ones((1, p.shape[1]), p.dtype)], axis=0)
    pe = pe.astype(jnp.bfloat16)

    labels = edge_ref[0, 0]
    iota = jax.lax.broadcasted_iota(jnp.int32, (labels.shape[0], _L), 1)
    onehot = (iota == labels).astype(jnp.bfloat16)

    part = jax.lax.dot_general(
        pe, onehot, (((1,), (0,)), ((), ())),
        preferred_element_type=jnp.float32)

    @pl.when(j == 0)
    def _init():
        stats_ref[0] = part

    @pl.when(j > 0)
    def _acc():
        stats_ref[0] += part


def _tail_kernel(stats_ref, adj_ref, out_ref, *, nb):
    b = pl.program_id(0)
    stats = stats_ref[0]
    sums = stats[:_C, :_K]
    counts = stats[_C:, :_K]
    safe = jnp.where(counts > 0, counts, jnp.ones_like(counts))
    mu = jnp.where(counts > 0, sums / safe, jnp.zeros_like(sums))
    nrm = jnp.sqrt(jnp.sum(mu * mu, axis=0, keepdims=True))
    mun = mu / (nrm + 1e-6)
    gram = jax.lax.dot_general(
        mun, mun, (((0,), (0,)), ((), ())),
        preferred_element_type=jnp.float32,
        precision=jax.lax.Precision.HIGHEST)
    adj = adj_ref[0]
    num = jnp.sum(gram * adj, axis=1, keepdims=True)
    den = jnp.sum(adj, axis=1, keepdims=True) + 1e-6
    part = jnp.sum(num / den, axis=(0, 1), keepdims=True) / (nb * _K)

    @pl.when(b == 0)
    def _init():
        out_ref[...] = part

    @pl.when(b > 0)
    def _acc():
        out_ref[...] += part


def kernel(pred_sg_up, edge_v, adj):
    B, C, H, W = pred_sg_up.shape
    N = H * W
    bN = 8192
    nblk = N // bN
    seg = pred_sg_up.reshape(B, C, N)
    edge = edge_v.reshape(B, nblk, bN, 1)

    stats = pl.pallas_call(
        _seg_stats_kernel,
        grid=(B, nblk),
        in_specs=[
            pl.BlockSpec((1, C, bN), lambda b, j: (b, 0, j)),
            pl.BlockSpec((1, 1, bN, 1), lambda b, j: (b, j, 0, 0)),
        ],
        out_specs=pl.BlockSpec((1, C + 1, _L), lambda b, j: (b, 0, 0)),
        out_shape=jax.ShapeDtypeStruct((B, C + 1, _L), jnp.float32),
    )(seg, edge)

    loss = pl.pallas_call(
        functools.partial(_tail_kernel, nb=B),
        grid=(B,),
        in_specs=[
            pl.BlockSpec((1, C + 1, _L), lambda b: (b, 0, 0)),
            pl.BlockSpec((1, _K, _K), lambda b: (b, 0, 0)),
        ],
        out_specs=pl.BlockSpec((1, 1), lambda b: (0, 0)),
        out_shape=jax.ShapeDtypeStruct((1, 1), jnp.float32),
    )(stats, adj)
    return loss[0, 0]


# speedup vs baseline: 8.2899x; 1.9961x over previous
import functools

import jax
import jax.numpy as jnp
from jax.experimental import pallas as pl

_C = 21
_L = 256
_K = 252


def _seg_stats_kernel(seg_ref, edge_ref, stats_ref):
    j = pl.program_id(1)
    x = seg_ref[0]
    m = jnp.max(x, axis=0, keepdims=True)
    e = jnp.exp(x - m)
    s = jnp.sum(e, axis=0, keepdims=True)
    p = e / s
    pe = jnp.concatenate([p, jnp.ones((1, p.shape[1]), p.dtype)], axis=0)
    pe = pe.astype(jnp.bfloat16)

    labels = jnp.swapaxes(edge_ref[0, 0], 0, 1)
    iota = jax.lax.broadcasted_iota(jnp.int32, (labels.shape[0], _L), 1)
    onehot = (iota == labels).astype(jnp.bfloat16)

    part = jax.lax.dot_general(
        pe, onehot, (((1,), (0,)), ((), ())),
        preferred_element_type=jnp.float32)

    @pl.when(j == 0)
    def _init():
        stats_ref[0] = part

    @pl.when(j > 0)
    def _acc():
        stats_ref[0] += part


def _tail_kernel(stats_ref, adj_ref, out_ref, *, nb):
    b = pl.program_id(0)
    stats = stats_ref[0]
    sums = stats[:_C, :_K]
    counts = stats[_C:, :_K]
    safe = jnp.where(counts > 0, counts, jnp.ones_like(counts))
    mu = jnp.where(counts > 0, sums / safe, jnp.zeros_like(sums))
    nrm = jnp.sqrt(jnp.sum(mu * mu, axis=0, keepdims=True))
    mun = mu / (nrm + 1e-6)
    gram = jax.lax.dot_general(
        mun, mun, (((0,), (0,)), ((), ())),
        preferred_element_type=jnp.float32,
        precision=jax.lax.Precision.HIGHEST)
    adj = adj_ref[0]
    num = jnp.sum(gram * adj, axis=1, keepdims=True)
    den = jnp.sum(adj, axis=1, keepdims=True) + 1e-6
    part = jnp.sum(num / den, axis=(0, 1), keepdims=True) / (nb * _K)

    @pl.when(b == 0)
    def _init():
        out_ref[...] = part

    @pl.when(b > 0)
    def _acc():
        out_ref[...] += part


def kernel(pred_sg_up, edge_v, adj):
    B, C, H, W = pred_sg_up.shape
    N = H * W
    bN = 8192
    nblk = N // bN
    seg = pred_sg_up.reshape(B, C, N)
    edge = edge_v.reshape(B, nblk, 1, bN)

    stats = pl.pallas_call(
        _seg_stats_kernel,
        grid=(B, nblk),
        in_specs=[
            pl.BlockSpec((1, C, bN), lambda b, j: (b, 0, j)),
            pl.BlockSpec((1, 1, 1, bN), lambda b, j: (b, j, 0, 0)),
        ],
        out_specs=pl.BlockSpec((1, C + 1, _L), lambda b, j: (b, 0, 0)),
        out_shape=jax.ShapeDtypeStruct((B, C + 1, _L), jnp.float32),
    )(seg, edge)

    loss = pl.pallas_call(
        functools.partial(_tail_kernel, nb=B),
        grid=(B,),
        in_specs=[
            pl.BlockSpec((1, C + 1, _L), lambda b: (b, 0, 0)),
            pl.BlockSpec((1, _K, _K), lambda b: (b, 0, 0)),
        ],
        out_specs=pl.BlockSpec((1, 1), lambda b: (0, 0)),
        out_shape=jax.ShapeDtypeStruct((1, 1), jnp.float32),
    )(stats, adj)
    return loss[0, 0]


# speedup vs baseline: 11.7759x; 1.4205x over previous
import functools

import jax
import jax.numpy as jnp
from jax.experimental import pallas as pl

_C = 21
_L = 256
_K = 252


def _seg_stats_kernel(seg_ref, edge_ref, stats_ref):
    j = pl.program_id(1)
    x = seg_ref[0]
    m = jnp.max(x, axis=0, keepdims=True)
    e = jnp.exp(x - m)
    s = jnp.sum(e, axis=0, keepdims=True)
    p = e / s
    pe = jnp.concatenate([p, jnp.ones((1, p.shape[1]), p.dtype)], axis=0)
    pe = pe.astype(jnp.bfloat16)

    labels = edge_ref[0, 0]
    iota = jax.lax.broadcasted_iota(jnp.int32, (_L, labels.shape[1]), 0)
    onehot_t = (iota == labels).astype(jnp.bfloat16)

    part = jax.lax.dot_general(
        pe, onehot_t, (((1,), (1,)), ((), ())),
        preferred_element_type=jnp.float32)

    @pl.when(j == 0)
    def _init():
        stats_ref[0] = part

    @pl.when(j > 0)
    def _acc():
        stats_ref[0] += part


def _tail_kernel(stats_ref, adj_ref, out_ref, *, nb):
    b = pl.program_id(0)
    stats = stats_ref[0]
    sums = stats[:_C, :_K]
    counts = stats[_C:, :_K]
    safe = jnp.where(counts > 0, counts, jnp.ones_like(counts))
    mu = jnp.where(counts > 0, sums / safe, jnp.zeros_like(sums))
    nrm = jnp.sqrt(jnp.sum(mu * mu, axis=0, keepdims=True))
    mun = mu / (nrm + 1e-6)
    gram = jax.lax.dot_general(
        mun, mun, (((0,), (0,)), ((), ())),
        preferred_element_type=jnp.float32,
        precision=jax.lax.Precision.HIGHEST)
    adj = adj_ref[0]
    num = jnp.sum(gram * adj, axis=1, keepdims=True)
    den = jnp.sum(adj, axis=1, keepdims=True) + 1e-6
    part = jnp.sum(num / den, axis=(0, 1), keepdims=True) / (nb * _K)

    @pl.when(b == 0)
    def _init():
        out_ref[...] = part

    @pl.when(b > 0)
    def _acc():
        out_ref[...] += part


def kernel(pred_sg_up, edge_v, adj):
    B, C, H, W = pred_sg_up.shape
    N = H * W
    bN = 8192
    nblk = N // bN
    seg = pred_sg_up.reshape(B, C, N)
    edge = edge_v.reshape(B, nblk, 1, bN)

    stats = pl.pallas_call(
        _seg_stats_kernel,
        grid=(B, nblk),
        in_specs=[
            pl.BlockSpec((1, C, bN), lambda b, j: (b, 0, j)),
            pl.BlockSpec((1, 1, 1, bN), lambda b, j: (b, j, 0, 0)),
        ],
        out_specs=pl.BlockSpec((1, C + 1, _L), lambda b, j: (b, 0, 0)),
        out_shape=jax.ShapeDtypeStruct((B, C + 1, _L), jnp.float32),
    )(seg, edge)

    loss = pl.pallas_call(
        functools.partial(_tail_kernel, nb=B),
        grid=(B,),
        in_specs=[
            pl.BlockSpec((1, C + 1, _L), lambda b: (b, 0, 0)),
            pl.BlockSpec((1, _K, _K), lambda b: (b, 0, 0)),
        ],
        out_specs=pl.BlockSpec((1, 1), lambda b: (0, 0)),
        out_shape=jax.ShapeDtypeStruct((1, 1), jnp.float32),
    )(stats, adj)
    return loss[0, 0]


# speedup vs baseline: 12.1665x; 1.0332x over previous
import functools

import jax
import jax.numpy as jnp
from jax.experimental import pallas as pl

_C = 21
_L = 256
_K = 252


def _seg_stats_kernel(seg_ref, edge_ref, stats_ref):
    j = pl.program_id(1)
    x = seg_ref[0]
    m = jnp.max(x, axis=0, keepdims=True)
    e = jnp.exp(x - m)
    s = jnp.sum(e, axis=0, keepdims=True)
    p = e / s
    pe = jnp.concatenate([p, jnp.ones((1, p.shape[1]), p.dtype)], axis=0)
    pe = pe.astype(jnp.bfloat16)

    labels = edge_ref[0, 0]
    iota = jax.lax.broadcasted_iota(jnp.int32, (_L, labels.shape[1]), 0)
    onehot_t = (iota == labels).astype(jnp.bfloat16)

    part = jax.lax.dot_general(
        pe, onehot_t, (((1,), (1,)), ((), ())),
        preferred_element_type=jnp.float32)

    @pl.when(j == 0)
    def _init():
        stats_ref[0] = part

    @pl.when(j > 0)
    def _acc():
        stats_ref[0] += part


def _tail_kernel(stats_ref, adj_ref, out_ref, *, nb):
    b = pl.program_id(0)
    stats = stats_ref[0]
    sums = stats[:_C, :_K]
    counts = stats[_C:, :_K]
    safe = jnp.where(counts > 0, counts, jnp.ones_like(counts))
    mu = jnp.where(counts > 0, sums / safe, jnp.zeros_like(sums))
    nrm = jnp.sqrt(jnp.sum(mu * mu, axis=0, keepdims=True))
    mun = mu / (nrm + 1e-6)
    gram = jax.lax.dot_general(
        mun, mun, (((0,), (0,)), ((), ())),
        preferred_element_type=jnp.float32,
        precision=jax.lax.Precision.HIGHEST)
    adj = adj_ref[0]
    num = jnp.sum(gram * adj, axis=1, keepdims=True)
    den = jnp.sum(adj, axis=1, keepdims=True) + 1e-6
    part = jnp.sum(num / den, axis=(0, 1), keepdims=True) / (nb * _K)

    @pl.when(b == 0)
    def _init():
        out_ref[...] = part

    @pl.when(b > 0)
    def _acc():
        out_ref[...] += part


def kernel(pred_sg_up, edge_v, adj):
    B, C, H, W = pred_sg_up.shape
    N = H * W
    bN = 16384
    nblk = N // bN
    seg = pred_sg_up.reshape(B, C, N)
    edge = edge_v.reshape(B, nblk, 1, bN)

    stats = pl.pallas_call(
        _seg_stats_kernel,
        grid=(B, nblk),
        in_specs=[
            pl.BlockSpec((1, C, bN), lambda b, j: (b, 0, j)),
            pl.BlockSpec((1, 1, 1, bN), lambda b, j: (b, j, 0, 0)),
        ],
        out_specs=pl.BlockSpec((1, C + 1, _L), lambda b, j: (b, 0, 0)),
        out_shape=jax.ShapeDtypeStruct((B, C + 1, _L), jnp.float32),
    )(seg, edge)

    loss = pl.pallas_call(
        functools.partial(_tail_kernel, nb=B),
        grid=(B,),
        in_specs=[
            pl.BlockSpec((1, C + 1, _L), lambda b: (b, 0, 0)),
            pl.BlockSpec((1, _K, _K), lambda b: (b, 0, 0)),
        ],
        out_specs=pl.BlockSpec((1, 1), lambda b: (0, 0)),
        out_shape=jax.ShapeDtypeStruct((1, 1), jnp.float32),
    )(stats, adj)
    return loss[0, 0]


# speedup vs baseline: 20.7358x; 1.7043x over previous
import functools

import jax
import jax.numpy as jnp
from jax.experimental import pallas as pl

_C = 21
_L = 256
_K = 252


def _seg_stats_kernel(seg_ref, edge_ref, stats_ref):
    j = pl.program_id(1)
    x = seg_ref[0]
    m = jnp.max(x, axis=0, keepdims=True)
    e = jnp.exp(x - m)
    s = jnp.sum(e, axis=0, keepdims=True)
    p = e / s
    pe = jnp.concatenate(
        [p, jnp.ones((1,) + p.shape[1:], p.dtype)], axis=0)
    pe = pe.astype(jnp.bfloat16)
    nbh, w = pe.shape[1], pe.shape[2]
    pe2 = pe.reshape(pe.shape[0], nbh * w)

    labels = edge_ref[0].reshape(1, nbh * w)
    iota = jax.lax.broadcasted_iota(jnp.int32, (_L, nbh * w), 0)
    onehot_t = (iota == labels).astype(jnp.bfloat16)

    part = jax.lax.dot_general(
        pe2, onehot_t, (((1,), (1,)), ((), ())),
        preferred_element_type=jnp.float32)

    @pl.when(j == 0)
    def _init():
        stats_ref[0] = part

    @pl.when(j > 0)
    def _acc():
        stats_ref[0] += part


def _tail_kernel(stats_ref, adj_ref, out_ref, *, nb):
    b = pl.program_id(0)
    stats = stats_ref[0]
    sums = stats[:_C, :_K]
    counts = stats[_C:, :_K]
    safe = jnp.where(counts > 0, counts, jnp.ones_like(counts))
    mu = jnp.where(counts > 0, sums / safe, jnp.zeros_like(sums))
    nrm = jnp.sqrt(jnp.sum(mu * mu, axis=0, keepdims=True))
    mun = mu / (nrm + 1e-6)
    gram = jax.lax.dot_general(
        mun, mun, (((0,), (0,)), ((), ())),
        preferred_element_type=jnp.float32,
        precision=jax.lax.Precision.HIGHEST)
    adj = adj_ref[0]
    num = jnp.sum(gram * adj, axis=1, keepdims=True)
    den = jnp.sum(adj, axis=1, keepdims=True) + 1e-6
    part = jnp.sum(num / den, axis=(0, 1), keepdims=True) / (nb * _K)

    @pl.when(b == 0)
    def _init():
        out_ref[...] = part

    @pl.when(b > 0)
    def _acc():
        out_ref[...] += part


def kernel(pred_sg_up, edge_v, adj):
    B, C, H, W = pred_sg_up.shape
    bH = 32
    nblk = H // bH

    stats = pl.pallas_call(
        _seg_stats_kernel,
        grid=(B, nblk),
        in_specs=[
            pl.BlockSpec((1, C, bH, W), lambda b, j: (b, 0, j, 0)),
            pl.BlockSpec((1, bH, W), lambda b, j: (b, j, 0)),
        ],
        out_specs=pl.BlockSpec((1, C + 1, _L), lambda b, j: (b, 0, 0)),
        out_shape=jax.ShapeDtypeStruct((B, C + 1, _L), jnp.float32),
    )(pred_sg_up, edge_v)

    loss = pl.pallas_call(
        functools.partial(_tail_kernel, nb=B),
        grid=(B,),
        in_specs=[
            pl.BlockSpec((1, C + 1, _L), lambda b: (b, 0, 0)),
            pl.BlockSpec((1, _K, _K), lambda b: (b, 0, 0)),
        ],
        out_specs=pl.BlockSpec((1, 1), lambda b: (0, 0)),
        out_shape=jax.ShapeDtypeStruct((1, 1), jnp.float32),
    )(stats, adj)
    return loss[0, 0]


# speedup vs baseline: 21.6127x; 1.0423x over previous
import functools

import jax
import jax.numpy as jnp
from jax.experimental import pallas as pl

_C = 21
_L = 256
_K = 252


def _seg_stats_kernel(seg_ref, edge_ref, stats_ref):
    j = pl.program_id(1)
    x = seg_ref[0]
    m = jnp.max(x, axis=0, keepdims=True)
    e = jnp.exp(x - m)
    s = jnp.sum(e, axis=0, keepdims=True)
    p = e / s
    pe = jnp.concatenate(
        [p, jnp.ones((1,) + p.shape[1:], p.dtype)], axis=0)
    pe = pe.astype(jnp.bfloat16)
    nbh, w = pe.shape[1], pe.shape[2]
    pe2 = pe.reshape(pe.shape[0], nbh * w)

    labels = edge_ref[0].reshape(1, nbh * w)
    iota = jax.lax.broadcasted_iota(jnp.int32, (_L, nbh * w), 0)
    onehot_t = (iota == labels).astype(jnp.bfloat16)

    part = jax.lax.dot_general(
        pe2, onehot_t, (((1,), (1,)), ((), ())),
        preferred_element_type=jnp.float32)

    @pl.when(j == 0)
    def _init():
        stats_ref[0] = part

    @pl.when(j > 0)
    def _acc():
        stats_ref[0] += part


def _tail_kernel(stats_ref, adj_ref, out_ref, *, nb):
    b = pl.program_id(0)
    stats = stats_ref[0]
    sums = stats[:_C, :_K]
    counts = stats[_C:, :_K]
    safe = jnp.where(counts > 0, counts, jnp.ones_like(counts))
    mu = jnp.where(counts > 0, sums / safe, jnp.zeros_like(sums))
    nrm = jnp.sqrt(jnp.sum(mu * mu, axis=0, keepdims=True))
    mun = mu / (nrm + 1e-6)
    gram = jax.lax.dot_general(
        mun, mun, (((0,), (0,)), ((), ())),
        preferred_element_type=jnp.float32,
        precision=jax.lax.Precision.HIGHEST)
    adj = adj_ref[0]
    num = jnp.sum(gram * adj, axis=1, keepdims=True)
    den = jnp.sum(adj, axis=1, keepdims=True) + 1e-6
    part = jnp.sum(num / den, axis=(0, 1), keepdims=True) / (nb * _K)

    @pl.when(b == 0)
    def _init():
        out_ref[...] = part

    @pl.when(b > 0)
    def _acc():
        out_ref[...] += part


def kernel(pred_sg_up, edge_v, adj):
    B, C, H, W = pred_sg_up.shape
    bH = 64
    nblk = H // bH

    stats = pl.pallas_call(
        _seg_stats_kernel,
        grid=(B, nblk),
        in_specs=[
            pl.BlockSpec((1, C, bH, W), lambda b, j: (b, 0, j, 0)),
            pl.BlockSpec((1, bH, W), lambda b, j: (b, j, 0)),
        ],
        out_specs=pl.BlockSpec((1, C + 1, _L), lambda b, j: (b, 0, 0)),
        out_shape=jax.ShapeDtypeStruct((B, C + 1, _L), jnp.float32),
    )(pred_sg_up, edge_v)

    loss = pl.pallas_call(
        functools.partial(_tail_kernel, nb=B),
        grid=(B,),
        in_specs=[
            pl.BlockSpec((1, C + 1, _L), lambda b: (b, 0, 0)),
            pl.BlockSpec((1, _K, _K), lambda b: (b, 0, 0)),
        ],
        out_specs=pl.BlockSpec((1, 1), lambda b: (0, 0)),
        out_shape=jax.ShapeDtypeStruct((1, 1), jnp.float32),
    )(stats, adj)
    return loss[0, 0]


# speedup vs baseline: 21.9797x; 1.0170x over previous
import functools

import jax
import jax.numpy as jnp
from jax.experimental import pallas as pl

_C = 21
_L = 256
_K = 252


def _seg_stats_kernel(seg_ref, edge_ref, stats_ref):
    j = pl.program_id(1)
    x = seg_ref[0]
    m = jnp.max(x, axis=0, keepdims=True)
    e = jnp.exp(x - m)
    s = jnp.sum(e, axis=0, keepdims=True)
    p = e / s
    pe = jnp.concatenate(
        [p, jnp.ones((1,) + p.shape[1:], p.dtype)], axis=0)
    pe = pe.astype(jnp.bfloat16)
    nbh, w = pe.shape[1], pe.shape[2]
    pe2 = pe.reshape(pe.shape[0], nbh * w)

    labels = edge_ref[0].reshape(1, nbh * w)
    iota = jax.lax.broadcasted_iota(jnp.int32, (_L, nbh * w), 0)
    onehot_t = (iota == labels).astype(jnp.bfloat16)

    part = jax.lax.dot_general(
        pe2, onehot_t, (((1,), (1,)), ((), ())),
        preferred_element_type=jnp.float32)

    @pl.when(j == 0)
    def _init():
        stats_ref[0] = part

    @pl.when(j > 0)
    def _acc():
        stats_ref[0] += part


def _tail_kernel(stats_ref, adj_ref, out_ref, *, nb):
    b = pl.program_id(0)
    stats = stats_ref[0]
    sums = stats[:_C, :_K]
    counts = stats[_C:, :_K]
    safe = jnp.where(counts > 0, counts, jnp.ones_like(counts))
    mu = jnp.where(counts > 0, sums / safe, jnp.zeros_like(sums))
    nrm = jnp.sqrt(jnp.sum(mu * mu, axis=0, keepdims=True))
    mun = mu / (nrm + 1e-6)
    gram = jax.lax.dot_general(
        mun, mun, (((0,), (0,)), ((), ())),
        preferred_element_type=jnp.float32,
        precision=jax.lax.Precision.HIGHEST)
    adj = adj_ref[0]
    num = jnp.sum(gram * adj, axis=1, keepdims=True)
    den = jnp.sum(adj, axis=1, keepdims=True) + 1e-6
    part = jnp.sum(num / den, axis=(0, 1), keepdims=True) / (nb * _K)

    @pl.when(b == 0)
    def _init():
        out_ref[...] = part

    @pl.when(b > 0)
    def _acc():
        out_ref[...] += part


def kernel(pred_sg_up, edge_v, adj):
    B, C, H, W = pred_sg_up.shape
    bH = 128
    nblk = H // bH

    stats = pl.pallas_call(
        _seg_stats_kernel,
        grid=(B, nblk),
        in_specs=[
            pl.BlockSpec((1, C, bH, W), lambda b, j: (b, 0, j, 0)),
            pl.BlockSpec((1, bH, W), lambda b, j: (b, j, 0)),
        ],
        out_specs=pl.BlockSpec((1, C + 1, _L), lambda b, j: (b, 0, 0)),
        out_shape=jax.ShapeDtypeStruct((B, C + 1, _L), jnp.float32),
    )(pred_sg_up, edge_v)

    loss = pl.pallas_call(
        functools.partial(_tail_kernel, nb=B),
        grid=(B,),
        in_specs=[
            pl.BlockSpec((1, C + 1, _L), lambda b: (b, 0, 0)),
            pl.BlockSpec((1, _K, _K), lambda b: (b, 0, 0)),
        ],
        out_specs=pl.BlockSpec((1, 1), lambda b: (0, 0)),
        out_shape=jax.ShapeDtypeStruct((1, 1), jnp.float32),
    )(stats, adj)
    return loss[0, 0]
